# Initial kernel scaffold; baseline (speedup 1.0000x reference)
import jax
import jax.numpy as jnp
from jax.experimental import pallas as pl


def kernel(atom_representation, positions, positions_probe, cells, probe_edges, probe_edges_displacement, splits, W_lin1, W_fc1, b_fc1, W_fc2, b_fc2, W_out_s, W_out_v):
    raise NotImplementedError("write your pallas kernel here")


# speedup vs baseline: 2.3176x; 2.3176x over previous
import functools

import jax
import jax.numpy as jnp
import numpy as np
from jax import lax
from jax.experimental import pallas as pl
from jax.experimental.pallas import tpu as pltpu
from jax.experimental.pallas import tpu_sc as plsc

N = 10000
P = 10000
E = 160000
D = 128
DV = 32
NB = 10
CUTOFF = 4.0
HID = 64
INV_SQRT_NN = 0.25

NC = 2
NS = 16
NW = NC * NS

E_PAD = 163840
CHUNK = 128
P_ACC = 10016
DUMMY = P

BE = 640
BP = 1000

INV_STEP = float((NB - 1) / CUTOFF)

def _basis_consts():
    rs = np.linspace(0.0, CUTOFF, 4001, dtype=np.float32)[1:]
    values = np.linspace(0.0, CUTOFF, NB, dtype=np.float32)
    step = values[1] - values[0]
    diff = (rs[:, None] - values[None, :]) / step
    bs = np.exp(-diff.astype(np.float32) ** 2) / 1.12
    mean = bs.mean(axis=0, dtype=np.float64).astype(np.float32)
    std = bs.std(axis=0, ddof=1, dtype=np.float64).astype(np.float32)
    out = np.zeros((8, 16), dtype=np.float32)
    out[0, :NB] = values
    out[1, :NB] = 1.0 / std
    out[2, :NB] = mean
    return out

_BCONST = _basis_consts()


def _lin1_body(x_ref, w_ref, o_ref):
    o_ref[...] = jnp.dot(x_ref[...], w_ref[...], preferred_element_type=jnp.float32)


def _lin1(atom, W):
    return pl.pallas_call(
        _lin1_body,
        grid=(10,),
        in_specs=[
            pl.BlockSpec((N // 10, D), lambda i: (i, 0)),
            pl.BlockSpec((D, D), lambda i: (0, 0)),
        ],
        out_specs=pl.BlockSpec((N // 10, D), lambda i: (i, 0)),
        out_shape=jax.ShapeDtypeStruct((N, D), jnp.float32),
    )(atom, W)


def _gather_body(src_hbm, dst_hbm, sender_hbm, pos_hbm, ppos_hbm,
                 gath_hbm, psrc_hbm, pdst_hbm,
                 idx_s, idx_d, rows, rp, rq, sem1, sem2, sem3):
    wid = lax.axis_index("s") * NC + lax.axis_index("c")
    base = wid * (E_PAD // NW)

    def step(i, _):
        off = base + i * CHUNK
        pltpu.sync_copy(src_hbm.at[pl.ds(off, CHUNK)], idx_s)
        pltpu.sync_copy(dst_hbm.at[pl.ds(off, CHUNK)], idx_d)
        c1 = pltpu.async_copy(sender_hbm.at[idx_s], rows, sem1)
        c2 = pltpu.async_copy(pos_hbm.at[idx_s], rp, sem2)
        c3 = pltpu.async_copy(ppos_hbm.at[idx_d], rq, sem3)
        c1.wait()
        c2.wait()
        c3.wait()
        pltpu.sync_copy(rows, gath_hbm.at[pl.ds(off, CHUNK)])
        pltpu.sync_copy(rp, psrc_hbm.at[pl.ds(off, CHUNK)])
        pltpu.sync_copy(rq, pdst_hbm.at[pl.ds(off, CHUNK)])
        return 0

    lax.fori_loop(0, E_PAD // NW // CHUNK, step, 0)


def _gather(src, dst, sender, pospad, ppospad):
    mesh = plsc.VectorSubcoreMesh(core_axis_name="c", subcore_axis_name="s",
                                  num_cores=NC, num_subcores=NS)
    f = pl.kernel(
        _gather_body,
        out_type=(
            jax.ShapeDtypeStruct((E_PAD, D), jnp.float32),
            jax.ShapeDtypeStruct((E_PAD, 16), jnp.float32),
            jax.ShapeDtypeStruct((E_PAD, 16), jnp.float32),
        ),
        mesh=mesh,
        scratch_types=[
            pltpu.VMEM((CHUNK,), jnp.int32),
            pltpu.VMEM((CHUNK,), jnp.int32),
            pltpu.VMEM((CHUNK, D), jnp.float32),
            pltpu.VMEM((CHUNK, 16), jnp.float32),
            pltpu.VMEM((CHUNK, 16), jnp.float32),
            pltpu.SemaphoreType.DMA,
            pltpu.SemaphoreType.DMA,
            pltpu.SemaphoreType.DMA,
        ],
        compiler_params=pltpu.CompilerParams(use_tc_tiling_on_sc=False),
    )
    return f(src, dst, sender, pospad, ppospad)


def _edge_body(gath_ref, psrc_ref, pdst_ref, ped_ref, c_ref, bc_ref,
               w1_ref, b1_ref, w2s_ref, b2s_ref, w2v_ref, b2v_ref,
               ms_ref, mv_ref):
    g = gath_ref[...]
    disp = jnp.dot(ped_ref[...], c_ref[...], preferred_element_type=jnp.float32)
    vec = pdst_ref[...] - psrc_ref[...] - disp
    l2 = jnp.sum(vec * vec, axis=1, keepdims=True) + 1e-12
    length = jnp.sqrt(l2)
    unit = vec / length

    values = bc_ref[0:1, :]
    inv_std = bc_ref[1:2, :]
    mean = bc_ref[2:3, :]
    diff = (length - values) * INV_STEP
    basis = jnp.exp(-diff * diff) * (1.0 / 1.12)
    bn = (basis - mean) * inv_std

    h = jnp.dot(bn, w1_ref[...], preferred_element_type=jnp.float32) + b1_ref[...]
    h = h * (1.0 / (1.0 + jnp.exp(-h)))
    ws = jnp.dot(h, w2s_ref[...], preferred_element_type=jnp.float32) + b2s_ref[...]
    wv = jnp.dot(h, w2v_ref[...], preferred_element_type=jnp.float32) + b2v_ref[...]

    ms_ref[...] = ws * g
    m = wv * g[:, :DV]
    ux = unit[:, 0:1]
    uy = unit[:, 1:2]
    uz = unit[:, 2:3]
    mv_ref[...] = jnp.concatenate(
        [m * ux, m * uy, m * uz, jnp.zeros_like(m)], axis=1)


def _edge_stage(gath, psrc, pdst, ped16, c16, bconst, w1, b1, w2s, b2s, w2v, b2v):
    nb = E_PAD // BE
    blk = lambda r, c: pl.BlockSpec((r, c), lambda i: (i, 0))
    full = lambda r, c: pl.BlockSpec((r, c), lambda i: (0, 0))
    return pl.pallas_call(
        _edge_body,
        grid=(nb,),
        in_specs=[
            blk(BE, D), blk(BE, 16), blk(BE, 16), blk(BE, 16),
            full(16, 16), full(8, 16),
            full(16, HID), full(1, HID),
            full(HID, D), full(1, D),
            full(HID, DV), full(1, DV),
        ],
        out_specs=[blk(BE, D), blk(BE, D)],
        out_shape=[
            jax.ShapeDtypeStruct((E_PAD, D), jnp.float32),
            jax.ShapeDtypeStruct((E_PAD, D), jnp.float32),
        ],
    )(gath, psrc, pdst, ped16, c16, bconst, w1, b1, w2s, b2s, w2v, b2v)


def _scatter_body(dsts_hbm, ms_hbm, mv_hbm, zero_hbm,
                  outs_hbm, outv_hbm, idx, rows, acc, sem):
    c = lax.axis_index("c")
    s = lax.axis_index("s")

    @pl.when(s == 0)
    def _():
        pltpu.sync_copy(zero_hbm, acc)

    plsc.subcore_barrier()

    base = s * (E_PAD // NS)

    def step(i, _):
        off = base + i * CHUNK
        pltpu.sync_copy(dsts_hbm.at[pl.ds(off, CHUNK)], idx)

        @pl.when(c == 0)
        def _():
            pltpu.sync_copy(ms_hbm.at[pl.ds(off, CHUNK)], rows)

        @pl.when(c == 1)
        def _():
            pltpu.sync_copy(mv_hbm.at[pl.ds(off, CHUNK)], rows)

        pltpu.sync_copy(rows, acc.at[idx], add=True)
        return 0

    lax.fori_loop(0, E_PAD // NS // CHUNK, step, 0)
    plsc.subcore_barrier()

    @pl.when((s == 0) & (c == 0))
    def _():
        pltpu.sync_copy(acc.at[pl.ds(0, P)], outs_hbm)

    @pl.when((s == 0) & (c == 1))
    def _():
        pltpu.sync_copy(acc.at[pl.ds(0, P)], outv_hbm)


def _scatter(dsts, ms, mv, zero):
    mesh = plsc.VectorSubcoreMesh(core_axis_name="c", subcore_axis_name="s",
                                  num_cores=NC, num_subcores=NS)
    f = pl.kernel(
        _scatter_body,
        out_type=(
            jax.ShapeDtypeStruct((P, D), jnp.float32),
            jax.ShapeDtypeStruct((P, D), jnp.float32),
        ),
        mesh=mesh,
        scratch_types=[
            pltpu.VMEM((CHUNK,), jnp.int32),
            pltpu.VMEM((CHUNK, D), jnp.float32),
            pltpu.VMEM_SHARED((P_ACC, D), jnp.float32),
            pltpu.SemaphoreType.DMA,
        ],
    )
    return f(dsts, ms, mv, zero)


def _readout_body(s_ref, v_ref, wos_ref, wov_ref, o_ref):
    ps = s_ref[...] * INV_SQRT_NN
    vx = v_ref[:, 0:DV] * INV_SQRT_NN
    vy = v_ref[:, DV:2 * DV] * INV_SQRT_NN
    vz = v_ref[:, 2 * DV:3 * DV] * INV_SQRT_NN
    vnorm = jnp.sqrt(vx * vx + vy * vy + vz * vz + 1e-12)
    o_ref[...] = (jnp.dot(ps, wos_ref[...], preferred_element_type=jnp.float32)
                  + jnp.dot(vnorm, wov_ref[...], preferred_element_type=jnp.float32))


def _readout(accs, accv, wos, wov):
    return pl.pallas_call(
        _readout_body,
        grid=(P // BP,),
        in_specs=[
            pl.BlockSpec((BP, D), lambda i: (i, 0)),
            pl.BlockSpec((BP, D), lambda i: (i, 0)),
            pl.BlockSpec((D, D), lambda i: (0, 0)),
            pl.BlockSpec((DV, D), lambda i: (0, 0)),
        ],
        out_specs=pl.BlockSpec((BP, D), lambda i: (i, 0)),
        out_shape=jax.ShapeDtypeStruct((P, D), jnp.float32),
    )(accs, accv, wos, wov)


def kernel(atom_representation, positions, positions_probe, cells, probe_edges,
           probe_edges_displacement, splits, W_lin1, W_fc1, b_fc1, W_fc2, b_fc2,
           W_out_s, W_out_v):
    pad = E_PAD - E
    src = jnp.pad(probe_edges[:, 0].astype(jnp.int32), (0, pad))
    dst = probe_edges[:, 1].astype(jnp.int32)
    dst_g = jnp.pad(dst, (0, pad))
    dst_s = jnp.pad(dst, (0, pad), constant_values=DUMMY)

    pospad = jnp.pad(positions, ((0, 0), (0, 13)))
    ppospad = jnp.pad(positions_probe, ((0, 0), (0, 13)))
    ped16 = jnp.pad(probe_edges_displacement, ((0, pad), (0, 13)))
    c16 = jnp.pad(cells[0], ((0, 13), (0, 13)))
    bconst = jnp.asarray(_BCONST)

    w1 = jnp.pad(W_fc1, ((0, 6), (0, 0)))
    b1 = b_fc1[None, :]
    w2s = W_fc2[:, :D]
    b2s = b_fc2[None, :D]
    w2v = W_fc2[:, D:]
    b2v = b_fc2[None, D:]
    zero = jnp.zeros((P_ACC, D), jnp.float32)

    sender = _lin1(atom_representation, W_lin1)
    gath, psrc, pdst = _gather(src, dst_g, sender, pospad, ppospad)
    ms, mv = _edge_stage(gath, psrc, pdst, ped16, c16, bconst,
                         w1, b1, w2s, b2s, w2v, b2v)
    accs, accv = _scatter(dst_s, ms, mv, zero)
    return _readout(accs, accv, W_out_s, W_out_v)


# speedup vs baseline: 2.6081x; 1.1253x over previous
import functools

import jax
import jax.numpy as jnp
import numpy as np
from jax import lax
from jax.experimental import pallas as pl
from jax.experimental.pallas import tpu as pltpu
from jax.experimental.pallas import tpu_sc as plsc

N = 10000
P = 10000
E = 160000
D = 128
DV = 32
NB = 10
CUTOFF = 4.0
HID = 64
INV_SQRT_NN = 0.25

NC = 2
NS = 16
NW = NC * NS

E_PAD = 163840
CHUNK = 128
P_ACC = 10016
DUMMY = P

BE = 640
BP = 1000

INV_STEP = float((NB - 1) / CUTOFF)

def _basis_consts():
    rs = np.linspace(0.0, CUTOFF, 4001, dtype=np.float32)[1:]
    values = np.linspace(0.0, CUTOFF, NB, dtype=np.float32)
    step = values[1] - values[0]
    diff = (rs[:, None] - values[None, :]) / step
    bs = np.exp(-diff.astype(np.float32) ** 2) / 1.12
    mean = bs.mean(axis=0, dtype=np.float64).astype(np.float32)
    std = bs.std(axis=0, ddof=1, dtype=np.float64).astype(np.float32)
    out = np.zeros((8, 16), dtype=np.float32)
    out[0, :NB] = values
    out[1, :NB] = 1.0 / std
    out[2, :NB] = mean
    return out

_BCONST = _basis_consts()


def _lin1_body(x_ref, w_ref, o_ref):
    o_ref[...] = jnp.dot(x_ref[...], w_ref[...], preferred_element_type=jnp.float32)


def _lin1(atom, W):
    return pl.pallas_call(
        _lin1_body,
        grid=(10,),
        in_specs=[
            pl.BlockSpec((N // 10, D), lambda i: (i, 0)),
            pl.BlockSpec((D, D), lambda i: (0, 0)),
        ],
        out_specs=pl.BlockSpec((N // 10, D), lambda i: (i, 0)),
        out_shape=jax.ShapeDtypeStruct((N, D), jnp.float32),
    )(atom, W)


NCHUNK = E_PAD // NW // CHUNK


def _gather_body(src_hbm, dst_hbm, sender_hbm, pos_hbm, ppos_hbm,
                 gath_hbm, psrc_hbm, pdst_hbm,
                 idx_s, idx_d, rows0, rows1, rp0, rp1, rq0, rq1,
                 isem, g0, g1, w0, w1):
    wid = lax.axis_index("s") * NC + lax.axis_index("c")
    base = wid * (E_PAD // NW)

    pltpu.async_copy(src_hbm.at[wid], idx_s, isem).wait()
    pltpu.async_copy(dst_hbm.at[wid], idx_d, isem).wait()

    def gathers(i, rows, rp, rq, sem):
        c1 = pltpu.async_copy(sender_hbm.at[idx_s.at[i]], rows, sem)
        c2 = pltpu.async_copy(pos_hbm.at[idx_s.at[i]], rp, sem)
        c3 = pltpu.async_copy(ppos_hbm.at[idx_d.at[i]], rq, sem)
        return c1, c2, c3

    def writes(i, rows, rp, rq, sem):
        off = base + i * CHUNK
        pltpu.async_copy(rows, gath_hbm.at[pl.ds(off, CHUNK)], sem)
        pltpu.async_copy(rp, psrc_hbm.at[pl.ds(off, CHUNK)], sem)
        pltpu.async_copy(rq, pdst_hbm.at[pl.ds(off, CHUNK)], sem)

    def drain_writes(rows, rp, rq, sem):
        pltpu.make_async_copy(rows, gath_hbm.at[pl.ds(base, CHUNK)], sem).wait()
        pltpu.make_async_copy(rp, psrc_hbm.at[pl.ds(base, CHUNK)], sem).wait()
        pltpu.make_async_copy(rq, pdst_hbm.at[pl.ds(base, CHUNK)], sem).wait()

    def drain(cs):
        for c in cs:
            c.wait()

    def step(j, _):
        a = 2 * j
        b = a + 1

        @pl.when(j > 0)
        def _():
            drain_writes(rows0, rp0, rq0, w0)
            drain_writes(rows1, rp1, rq1, w1)

        ca = gathers(a, rows0, rp0, rq0, g0)
        cb = gathers(b, rows1, rp1, rq1, g1)
        drain(ca)
        writes(a, rows0, rp0, rq0, w0)
        drain(cb)
        writes(b, rows1, rp1, rq1, w1)
        return 0

    lax.fori_loop(0, NCHUNK // 2, step, 0)
    drain_writes(rows0, rp0, rq0, w0)
    drain_writes(rows1, rp1, rq1, w1)


def _gather(src, dst, sender, pospad, ppospad):
    mesh = plsc.VectorSubcoreMesh(core_axis_name="c", subcore_axis_name="s",
                                  num_cores=NC, num_subcores=NS)
    f = pl.kernel(
        _gather_body,
        out_type=(
            jax.ShapeDtypeStruct((E_PAD, D), jnp.float32),
            jax.ShapeDtypeStruct((E_PAD, 16), jnp.float32),
            jax.ShapeDtypeStruct((E_PAD, 16), jnp.float32),
        ),
        mesh=mesh,
        scratch_types=[
            pltpu.VMEM((NCHUNK, CHUNK), jnp.int32),
            pltpu.VMEM((NCHUNK, CHUNK), jnp.int32),
            pltpu.VMEM((CHUNK, D), jnp.float32),
            pltpu.VMEM((CHUNK, D), jnp.float32),
            pltpu.VMEM((CHUNK, 16), jnp.float32),
            pltpu.VMEM((CHUNK, 16), jnp.float32),
            pltpu.VMEM((CHUNK, 16), jnp.float32),
            pltpu.VMEM((CHUNK, 16), jnp.float32),
            pltpu.SemaphoreType.DMA,
            pltpu.SemaphoreType.DMA,
            pltpu.SemaphoreType.DMA,
            pltpu.SemaphoreType.DMA,
            pltpu.SemaphoreType.DMA,
        ],
        compiler_params=pltpu.CompilerParams(use_tc_tiling_on_sc=False),
    )
    return f(src, dst, sender, pospad, ppospad)


def _edge_body(gath_ref, psrc_ref, pdst_ref, ped_ref, c_ref, bc_ref,
               w1_ref, b1_ref, w2s_ref, b2s_ref, w2v_ref, b2v_ref,
               ms_ref, mv_ref):
    g = gath_ref[...]
    disp = jnp.dot(ped_ref[...], c_ref[...], preferred_element_type=jnp.float32)
    vec = pdst_ref[...] - psrc_ref[...] - disp
    l2 = jnp.sum(vec * vec, axis=1, keepdims=True) + 1e-12
    length = jnp.sqrt(l2)
    unit = vec / length

    values = bc_ref[0:1, :]
    inv_std = bc_ref[1:2, :]
    mean = bc_ref[2:3, :]
    diff = (length - values) * INV_STEP
    basis = jnp.exp(-diff * diff) * (1.0 / 1.12)
    bn = (basis - mean) * inv_std

    h = jnp.dot(bn, w1_ref[...], preferred_element_type=jnp.float32) + b1_ref[...]
    h = h * (1.0 / (1.0 + jnp.exp(-h)))
    ws = jnp.dot(h, w2s_ref[...], preferred_element_type=jnp.float32) + b2s_ref[...]
    wv = jnp.dot(h, w2v_ref[...], preferred_element_type=jnp.float32) + b2v_ref[...]

    ms_ref[...] = ws * g
    m = wv * g[:, :DV]
    ux = unit[:, 0:1]
    uy = unit[:, 1:2]
    uz = unit[:, 2:3]
    mv_ref[...] = jnp.concatenate(
        [m * ux, m * uy, m * uz, jnp.zeros_like(m)], axis=1)


def _edge_stage(gath, psrc, pdst, ped16, c16, bconst, w1, b1, w2s, b2s, w2v, b2v):
    nb = E_PAD // BE
    blk = lambda r, c: pl.BlockSpec((r, c), lambda i: (i, 0))
    full = lambda r, c: pl.BlockSpec((r, c), lambda i: (0, 0))
    return pl.pallas_call(
        _edge_body,
        grid=(nb,),
        in_specs=[
            blk(BE, D), blk(BE, 16), blk(BE, 16), blk(BE, 16),
            full(16, 16), full(8, 16),
            full(16, HID), full(1, HID),
            full(HID, D), full(1, D),
            full(HID, DV), full(1, DV),
        ],
        out_specs=[blk(BE, D), blk(BE, D)],
        out_shape=[
            jax.ShapeDtypeStruct((E_PAD, D), jnp.float32),
            jax.ShapeDtypeStruct((E_PAD, D), jnp.float32),
        ],
    )(gath, psrc, pdst, ped16, c16, bconst, w1, b1, w2s, b2s, w2v, b2v)


NSCHUNK = E_PAD // NS // CHUNK


def _scatter_body(dsts_hbm, ms_hbm, mv_hbm, zero_hbm,
                  outs_hbm, outv_hbm, idx, rows0, rows1, acc,
                  isem, l0, l1, a0, a1):
    c = lax.axis_index("c")
    s = lax.axis_index("s")

    @pl.when(s == 0)
    def _():
        pltpu.sync_copy(zero_hbm, acc)

    pltpu.async_copy(dsts_hbm.at[s], idx, isem).wait()

    plsc.subcore_barrier()

    base = s * (E_PAD // NS)

    def load(i, rows, sem):
        off = base + i * CHUNK

        @pl.when(c == 0)
        def _():
            pltpu.async_copy(ms_hbm.at[pl.ds(off, CHUNK)], rows, sem)

        @pl.when(c == 1)
        def _():
            pltpu.async_copy(mv_hbm.at[pl.ds(off, CHUNK)], rows, sem)

    def wait_load(rows, sem):
        pltpu.make_async_copy(ms_hbm.at[pl.ds(base, CHUNK)], rows, sem).wait()

    load(0, rows0, l0)
    load(1, rows1, l1)

    def step(j, _):
        a = 2 * j
        b = a + 1
        wait_load(rows0, l0)
        ca = pltpu.async_copy(rows0, acc.at[idx.at[a]], a0, add=True)
        wait_load(rows1, l1)
        cb = pltpu.async_copy(rows1, acc.at[idx.at[b]], a1, add=True)
        ca.wait()

        @pl.when(a + 2 < NSCHUNK)
        def _():
            load(a + 2, rows0, l0)

        cb.wait()

        @pl.when(b + 2 < NSCHUNK)
        def _():
            load(b + 2, rows1, l1)

        return 0

    lax.fori_loop(0, NSCHUNK // 2, step, 0)
    plsc.subcore_barrier()

    @pl.when((s == 0) & (c == 0))
    def _():
        pltpu.sync_copy(acc.at[pl.ds(0, P)], outs_hbm)

    @pl.when((s == 0) & (c == 1))
    def _():
        pltpu.sync_copy(acc.at[pl.ds(0, P)], outv_hbm)


def _scatter(dsts, ms, mv, zero):
    mesh = plsc.VectorSubcoreMesh(core_axis_name="c", subcore_axis_name="s",
                                  num_cores=NC, num_subcores=NS)
    f = pl.kernel(
        _scatter_body,
        out_type=(
            jax.ShapeDtypeStruct((P, D), jnp.float32),
            jax.ShapeDtypeStruct((P, D), jnp.float32),
        ),
        mesh=mesh,
        scratch_types=[
            pltpu.VMEM((NSCHUNK, CHUNK), jnp.int32),
            pltpu.VMEM((CHUNK, D), jnp.float32),
            pltpu.VMEM((CHUNK, D), jnp.float32),
            pltpu.VMEM_SHARED((P_ACC, D), jnp.float32),
            pltpu.SemaphoreType.DMA,
            pltpu.SemaphoreType.DMA,
            pltpu.SemaphoreType.DMA,
            pltpu.SemaphoreType.DMA,
            pltpu.SemaphoreType.DMA,
        ],
    )
    return f(dsts, ms, mv, zero)


def _readout_body(s_ref, v_ref, wos_ref, wov_ref, o_ref):
    ps = s_ref[...] * INV_SQRT_NN
    vx = v_ref[:, 0:DV] * INV_SQRT_NN
    vy = v_ref[:, DV:2 * DV] * INV_SQRT_NN
    vz = v_ref[:, 2 * DV:3 * DV] * INV_SQRT_NN
    vnorm = jnp.sqrt(vx * vx + vy * vy + vz * vz + 1e-12)
    o_ref[...] = (jnp.dot(ps, wos_ref[...], preferred_element_type=jnp.float32)
                  + jnp.dot(vnorm, wov_ref[...], preferred_element_type=jnp.float32))


def _readout(accs, accv, wos, wov):
    return pl.pallas_call(
        _readout_body,
        grid=(P // BP,),
        in_specs=[
            pl.BlockSpec((BP, D), lambda i: (i, 0)),
            pl.BlockSpec((BP, D), lambda i: (i, 0)),
            pl.BlockSpec((D, D), lambda i: (0, 0)),
            pl.BlockSpec((DV, D), lambda i: (0, 0)),
        ],
        out_specs=pl.BlockSpec((BP, D), lambda i: (i, 0)),
        out_shape=jax.ShapeDtypeStruct((P, D), jnp.float32),
    )(accs, accv, wos, wov)


def kernel(atom_representation, positions, positions_probe, cells, probe_edges,
           probe_edges_displacement, splits, W_lin1, W_fc1, b_fc1, W_fc2, b_fc2,
           W_out_s, W_out_v):
    pad = E_PAD - E
    src = jnp.pad(probe_edges[:, 0].astype(jnp.int32),
                  (0, pad)).reshape(NW, NCHUNK, CHUNK)
    dst = probe_edges[:, 1].astype(jnp.int32)
    dst_g = jnp.pad(dst, (0, pad)).reshape(NW, NCHUNK, CHUNK)
    dst_s = jnp.pad(dst, (0, pad),
                    constant_values=DUMMY).reshape(NS, NSCHUNK, CHUNK)

    pospad = jnp.pad(positions, ((0, 0), (0, 13)))
    ppospad = jnp.pad(positions_probe, ((0, 0), (0, 13)))
    ped16 = jnp.pad(probe_edges_displacement, ((0, pad), (0, 13)))
    c16 = jnp.pad(cells[0], ((0, 13), (0, 13)))
    bconst = jnp.asarray(_BCONST)

    w1 = jnp.pad(W_fc1, ((0, 6), (0, 0)))
    b1 = b_fc1[None, :]
    w2s = W_fc2[:, :D]
    b2s = b_fc2[None, :D]
    w2v = W_fc2[:, D:]
    b2v = b_fc2[None, D:]
    zero = jnp.zeros((P_ACC, D), jnp.float32)

    sender = _lin1(atom_representation, W_lin1)
    gath, psrc, pdst = _gather(src, dst_g, sender, pospad, ppospad)
    ms, mv = _edge_stage(gath, psrc, pdst, ped16, c16, bconst,
                         w1, b1, w2s, b2s, w2v, b2v)
    accs, accv = _scatter(dst_s, ms, mv, zero)
    return _readout(accs, accv, W_out_s, W_out_v)


# speedup vs baseline: 2.7591x; 1.0579x over previous
import functools

import jax
import jax.numpy as jnp
import numpy as np
from jax import lax
from jax.experimental import pallas as pl
from jax.experimental.pallas import tpu as pltpu
from jax.experimental.pallas import tpu_sc as plsc

N = 10000
P = 10000
E = 160000
D = 128
DV = 32
NB = 10
CUTOFF = 4.0
HID = 64
INV_SQRT_NN = 0.25

NC = 2
NS = 16
NW = NC * NS

E_PAD = 163840
CHUNK = 128
P_ACC = 10016
DUMMY = P

NSLAB = 2
SLAB = E_PAD // NSLAB
GCH = SLAB // NW // CHUNK
SCH = SLAB // NS // CHUNK

BE = 640
BP = 1000

INV_STEP = float((NB - 1) / CUTOFF)

def _basis_consts():
    rs = np.linspace(0.0, CUTOFF, 4001, dtype=np.float32)[1:]
    values = np.linspace(0.0, CUTOFF, NB, dtype=np.float32)
    step = values[1] - values[0]
    diff = (rs[:, None] - values[None, :]) / step
    bs = np.exp(-diff.astype(np.float32) ** 2) / 1.12
    mean = bs.mean(axis=0, dtype=np.float64).astype(np.float32)
    std = bs.std(axis=0, ddof=1, dtype=np.float64).astype(np.float32)
    out = np.zeros((8, 16), dtype=np.float32)
    out[0, :NB] = values
    out[1, :NB] = 1.0 / std
    out[2, :NB] = mean
    return out

_BCONST = _basis_consts()


def _sc_mesh():
    return plsc.VectorSubcoreMesh(core_axis_name="c", subcore_axis_name="s",
                                  num_cores=NC, num_subcores=NS)


def _lin1_body(x_ref, w_ref, o_ref):
    o_ref[...] = jnp.dot(x_ref[...], w_ref[...], preferred_element_type=jnp.float32)


def _lin1(atom, W):
    return pl.pallas_call(
        _lin1_body,
        grid=(10,),
        in_specs=[
            pl.BlockSpec((N // 10, D), lambda i: (i, 0)),
            pl.BlockSpec((D, D), lambda i: (0, 0)),
        ],
        out_specs=pl.BlockSpec((N // 10, D), lambda i: (i, 0)),
        out_shape=jax.ShapeDtypeStruct((N, D), jnp.float32),
    )(atom, W)


def _gather_body(src_hbm, dst_hbm, sender_hbm, pos_hbm, ppos_hbm,
                 gath_hbm, psrc_hbm, pdst_hbm,
                 idx_s, idx_d, rows0, rows1, rp0, rp1, rq0, rq1,
                 isem, g0, g1, w0, w1):
    wid = lax.axis_index("s") * NC + lax.axis_index("c")
    base = wid * (SLAB // NW)

    pltpu.async_copy(src_hbm.at[wid], idx_s, isem).wait()
    pltpu.async_copy(dst_hbm.at[wid], idx_d, isem).wait()

    def gathers(i, rows, rp, rq, sem):
        c1 = pltpu.async_copy(sender_hbm.at[idx_s.at[i]], rows, sem)
        c2 = pltpu.async_copy(pos_hbm.at[idx_s.at[i]], rp, sem)
        c3 = pltpu.async_copy(ppos_hbm.at[idx_d.at[i]], rq, sem)
        return c1, c2, c3

    def writes(i, rows, rp, rq, sem):
        off = base + i * CHUNK
        pltpu.async_copy(rows, gath_hbm.at[pl.ds(off, CHUNK)], sem)
        pltpu.async_copy(rp, psrc_hbm.at[pl.ds(off, CHUNK)], sem)
        pltpu.async_copy(rq, pdst_hbm.at[pl.ds(off, CHUNK)], sem)

    def drain_writes(rows, rp, rq, sem):
        pltpu.make_async_copy(rows, gath_hbm.at[pl.ds(base, CHUNK)], sem).wait()
        pltpu.make_async_copy(rp, psrc_hbm.at[pl.ds(base, CHUNK)], sem).wait()
        pltpu.make_async_copy(rq, pdst_hbm.at[pl.ds(base, CHUNK)], sem).wait()

    def drain(cs):
        for c in cs:
            c.wait()

    def step(j, _):
        a = 2 * j
        b = a + 1

        @pl.when(j > 0)
        def _():
            drain_writes(rows0, rp0, rq0, w0)
            drain_writes(rows1, rp1, rq1, w1)

        ca = gathers(a, rows0, rp0, rq0, g0)
        cb = gathers(b, rows1, rp1, rq1, g1)
        drain(ca)
        writes(a, rows0, rp0, rq0, w0)
        drain(cb)
        writes(b, rows1, rp1, rq1, w1)
        return 0

    lax.fori_loop(0, GCH // 2, step, 0)
    drain_writes(rows0, rp0, rq0, w0)
    drain_writes(rows1, rp1, rq1, w1)


def _gather(src, dst, sender, pospad, ppospad):
    f = pl.kernel(
        _gather_body,
        out_type=(
            jax.ShapeDtypeStruct((SLAB, D), jnp.float32),
            jax.ShapeDtypeStruct((SLAB, 16), jnp.float32),
            jax.ShapeDtypeStruct((SLAB, 16), jnp.float32),
        ),
        mesh=_sc_mesh(),
        scratch_types=[
            pltpu.VMEM((GCH, CHUNK), jnp.int32),
            pltpu.VMEM((GCH, CHUNK), jnp.int32),
            pltpu.VMEM((CHUNK, D), jnp.float32),
            pltpu.VMEM((CHUNK, D), jnp.float32),
            pltpu.VMEM((CHUNK, 16), jnp.float32),
            pltpu.VMEM((CHUNK, 16), jnp.float32),
            pltpu.VMEM((CHUNK, 16), jnp.float32),
            pltpu.VMEM((CHUNK, 16), jnp.float32),
            pltpu.SemaphoreType.DMA,
            pltpu.SemaphoreType.DMA,
            pltpu.SemaphoreType.DMA,
            pltpu.SemaphoreType.DMA,
            pltpu.SemaphoreType.DMA,
        ],
        compiler_params=pltpu.CompilerParams(use_tc_tiling_on_sc=False),
    )
    return f(src, dst, sender, pospad, ppospad)


def _edge_body(gath_ref, psrc_ref, pdst_ref, ped_ref, c_ref, bc_ref,
               w1_ref, b1_ref, w2s_ref, b2s_ref, w2v_ref, b2v_ref,
               ms_ref, mv_ref):
    g = gath_ref[...]
    disp = jnp.dot(ped_ref[...], c_ref[...], preferred_element_type=jnp.float32)
    vec = pdst_ref[...] - psrc_ref[...] - disp
    l2 = jnp.sum(vec * vec, axis=1, keepdims=True) + 1e-12
    length = jnp.sqrt(l2)
    unit = vec / length

    values = bc_ref[0:1, :]
    inv_std = bc_ref[1:2, :]
    mean = bc_ref[2:3, :]
    diff = (length - values) * INV_STEP
    basis = jnp.exp(-diff * diff) * (1.0 / 1.12)
    bn = (basis - mean) * inv_std

    h = jnp.dot(bn, w1_ref[...], preferred_element_type=jnp.float32) + b1_ref[...]
    h = h * (1.0 / (1.0 + jnp.exp(-h)))
    ws = jnp.dot(h, w2s_ref[...], preferred_element_type=jnp.float32) + b2s_ref[...]
    wv = jnp.dot(h, w2v_ref[...], preferred_element_type=jnp.float32) + b2v_ref[...]

    ms_ref[...] = ws * g
    m = wv * g[:, :DV]
    ux = unit[:, 0:1]
    uy = unit[:, 1:2]
    uz = unit[:, 2:3]
    mv_ref[...] = jnp.concatenate(
        [m * ux, m * uy, m * uz, jnp.zeros_like(m)], axis=1)


def _edge_stage(gath, psrc, pdst, ped16, c16, bconst, w1, b1, w2s, b2s, w2v, b2v):
    nb = SLAB // BE
    blk = lambda r, c: pl.BlockSpec((r, c), lambda i: (i, 0))
    full = lambda r, c: pl.BlockSpec((r, c), lambda i: (0, 0))
    return pl.pallas_call(
        _edge_body,
        grid=(nb,),
        in_specs=[
            blk(BE, D), blk(BE, 16), blk(BE, 16), blk(BE, 16),
            full(16, 16), full(8, 16),
            full(16, HID), full(1, HID),
            full(HID, D), full(1, D),
            full(HID, DV), full(1, DV),
        ],
        out_specs=[blk(BE, D), blk(BE, D)],
        out_shape=[
            jax.ShapeDtypeStruct((SLAB, D), jnp.float32),
            jax.ShapeDtypeStruct((SLAB, D), jnp.float32),
        ],
    )(gath, psrc, pdst, ped16, c16, bconst, w1, b1, w2s, b2s, w2v, b2v)


def _scatter_body(*refs):
    dsts_list = refs[0:NSLAB]
    ms_list = refs[NSLAB:2 * NSLAB]
    mv_list = refs[2 * NSLAB:3 * NSLAB]
    zero_hbm = refs[3 * NSLAB]
    outs_hbm = refs[3 * NSLAB + 1]
    outv_hbm = refs[3 * NSLAB + 2]
    idx, rows0, rows1, acc, isem, l0, l1, a0, a1 = refs[3 * NSLAB + 3:]

    c = lax.axis_index("c")
    s = lax.axis_index("s")

    @pl.when(s == 0)
    def _():
        pltpu.sync_copy(zero_hbm, acc)

    plsc.subcore_barrier()

    base = s * (SLAB // NS)

    for k in range(NSLAB):
        ms_hbm = ms_list[k]
        mv_hbm = mv_list[k]

        pltpu.async_copy(dsts_list[k].at[s], idx, isem).wait()

        def load(i, rows, sem):
            off = base + i * CHUNK

            @pl.when(c == 0)
            def _():
                pltpu.async_copy(ms_hbm.at[pl.ds(off, CHUNK)], rows, sem)

            @pl.when(c == 1)
            def _():
                pltpu.async_copy(mv_hbm.at[pl.ds(off, CHUNK)], rows, sem)

        def wait_load(rows, sem):
            pltpu.make_async_copy(ms_hbm.at[pl.ds(base, CHUNK)], rows, sem).wait()

        load(0, rows0, l0)
        load(1, rows1, l1)

        def step(j, _):
            a = 2 * j
            b = a + 1
            wait_load(rows0, l0)
            ca = pltpu.async_copy(rows0, acc.at[idx.at[a]], a0, add=True)
            wait_load(rows1, l1)
            cb = pltpu.async_copy(rows1, acc.at[idx.at[b]], a1, add=True)
            ca.wait()

            @pl.when(a + 2 < SCH)
            def _():
                load(a + 2, rows0, l0)

            cb.wait()

            @pl.when(b + 2 < SCH)
            def _():
                load(b + 2, rows1, l1)

            return 0

        lax.fori_loop(0, SCH // 2, step, 0)

    plsc.subcore_barrier()

    @pl.when((s == 0) & (c == 0))
    def _():
        pltpu.sync_copy(acc.at[pl.ds(0, P)], outs_hbm)

    @pl.when((s == 0) & (c == 1))
    def _():
        pltpu.sync_copy(acc.at[pl.ds(0, P)], outv_hbm)


def _scatter(dsts_list, ms_list, mv_list, zero):
    f = pl.kernel(
        _scatter_body,
        out_type=(
            jax.ShapeDtypeStruct((P, D), jnp.float32),
            jax.ShapeDtypeStruct((P, D), jnp.float32),
        ),
        mesh=_sc_mesh(),
        scratch_types=[
            pltpu.VMEM((SCH, CHUNK), jnp.int32),
            pltpu.VMEM((CHUNK, D), jnp.float32),
            pltpu.VMEM((CHUNK, D), jnp.float32),
            pltpu.VMEM_SHARED((P_ACC, D), jnp.float32),
            pltpu.SemaphoreType.DMA,
            pltpu.SemaphoreType.DMA,
            pltpu.SemaphoreType.DMA,
            pltpu.SemaphoreType.DMA,
            pltpu.SemaphoreType.DMA,
        ],
    )
    return f(*dsts_list, *ms_list, *mv_list, zero)


def _readout_body(s_ref, v_ref, wos_ref, wov_ref, o_ref):
    ps = s_ref[...] * INV_SQRT_NN
    vx = v_ref[:, 0:DV] * INV_SQRT_NN
    vy = v_ref[:, DV:2 * DV] * INV_SQRT_NN
    vz = v_ref[:, 2 * DV:3 * DV] * INV_SQRT_NN
    vnorm = jnp.sqrt(vx * vx + vy * vy + vz * vz + 1e-12)
    o_ref[...] = (jnp.dot(ps, wos_ref[...], preferred_element_type=jnp.float32)
                  + jnp.dot(vnorm, wov_ref[...], preferred_element_type=jnp.float32))


def _readout(accs, accv, wos, wov):
    return pl.pallas_call(
        _readout_body,
        grid=(P // BP,),
        in_specs=[
            pl.BlockSpec((BP, D), lambda i: (i, 0)),
            pl.BlockSpec((BP, D), lambda i: (i, 0)),
            pl.BlockSpec((D, D), lambda i: (0, 0)),
            pl.BlockSpec((DV, D), lambda i: (0, 0)),
        ],
        out_specs=pl.BlockSpec((BP, D), lambda i: (i, 0)),
        out_shape=jax.ShapeDtypeStruct((P, D), jnp.float32),
    )(accs, accv, wos, wov)


def kernel(atom_representation, positions, positions_probe, cells, probe_edges,
           probe_edges_displacement, splits, W_lin1, W_fc1, b_fc1, W_fc2, b_fc2,
           W_out_s, W_out_v):
    pad = E_PAD - E
    src = jnp.pad(probe_edges[:, 0].astype(jnp.int32), (0, pad))
    dst = probe_edges[:, 1].astype(jnp.int32)
    dst_g = jnp.pad(dst, (0, pad))
    dst_s = jnp.pad(dst, (0, pad), constant_values=DUMMY)

    src_sl = src.reshape(NSLAB, NW, GCH, CHUNK)
    dstg_sl = dst_g.reshape(NSLAB, NW, GCH, CHUNK)
    dsts_sl = dst_s.reshape(NSLAB, NS, SCH, CHUNK)

    pospad = jnp.pad(positions, ((0, 0), (0, 13)))
    ppospad = jnp.pad(positions_probe, ((0, 0), (0, 13)))
    ped16 = jnp.pad(probe_edges_displacement, ((0, pad), (0, 13)))
    c16 = jnp.pad(cells[0], ((0, 13), (0, 13)))
    bconst = jnp.asarray(_BCONST)

    w1 = jnp.pad(W_fc1, ((0, 6), (0, 0)))
    b1 = b_fc1[None, :]
    w2s = W_fc2[:, :D]
    b2s = b_fc2[None, :D]
    w2v = W_fc2[:, D:]
    b2v = b_fc2[None, D:]
    zero = jnp.zeros((P_ACC, D), jnp.float32)

    sender = _lin1(atom_representation, W_lin1)

    ms_list, mv_list, dsts_list = [], [], []
    for k in range(NSLAB):
        gath, psrc, pdst = _gather(src_sl[k], dstg_sl[k], sender, pospad, ppospad)
        ms, mv = _edge_stage(gath, psrc, pdst,
                             ped16[k * SLAB:(k + 1) * SLAB], c16, bconst,
                             w1, b1, w2s, b2s, w2v, b2v)
        ms_list.append(ms)
        mv_list.append(mv)
        dsts_list.append(dsts_sl[k])

    accs, accv = _scatter(dsts_list, ms_list, mv_list, zero)
    return _readout(accs, accv, W_out_s, W_out_v)


# speedup vs baseline: 2.8059x; 1.0170x over previous
import functools

import jax
import jax.numpy as jnp
import numpy as np
from jax import lax
from jax.experimental import pallas as pl
from jax.experimental.pallas import tpu as pltpu
from jax.experimental.pallas import tpu_sc as plsc

N = 10000
P = 10000
E = 160000
D = 128
DV = 32
NB = 10
CUTOFF = 4.0
HID = 64
INV_SQRT_NN = 0.25

NC = 2
NS = 16
NW = NC * NS

E_PAD = 163840
CHUNK = 128
P_ACC = 10016
DUMMY = P

NSLAB = 4
SLAB = E_PAD // NSLAB
GCH = SLAB // NW // CHUNK
SCH = SLAB // NS // CHUNK

BE = 640
BP = 1000

INV_STEP = float((NB - 1) / CUTOFF)

def _basis_consts():
    rs = np.linspace(0.0, CUTOFF, 4001, dtype=np.float32)[1:]
    values = np.linspace(0.0, CUTOFF, NB, dtype=np.float32)
    step = values[1] - values[0]
    diff = (rs[:, None] - values[None, :]) / step
    bs = np.exp(-diff.astype(np.float32) ** 2) / 1.12
    mean = bs.mean(axis=0, dtype=np.float64).astype(np.float32)
    std = bs.std(axis=0, ddof=1, dtype=np.float64).astype(np.float32)
    out = np.zeros((8, 16), dtype=np.float32)
    out[0, :NB] = values
    out[1, :NB] = 1.0 / std
    out[2, :NB] = mean
    return out

_BCONST = _basis_consts()


def _sc_mesh():
    return plsc.VectorSubcoreMesh(core_axis_name="c", subcore_axis_name="s",
                                  num_cores=NC, num_subcores=NS)


def _lin1_body(x_ref, w_ref, o_ref):
    o_ref[...] = jnp.dot(x_ref[...], w_ref[...], preferred_element_type=jnp.float32)


def _lin1(atom, W):
    return pl.pallas_call(
        _lin1_body,
        grid=(10,),
        in_specs=[
            pl.BlockSpec((N // 10, D), lambda i: (i, 0)),
            pl.BlockSpec((D, D), lambda i: (0, 0)),
        ],
        out_specs=pl.BlockSpec((N // 10, D), lambda i: (i, 0)),
        out_shape=jax.ShapeDtypeStruct((N, D), jnp.float32),
    )(atom, W)


def _gather_body(src_hbm, dst_hbm, sender_hbm, pos_hbm, ppos_hbm,
                 gath_hbm, psrc_hbm, pdst_hbm,
                 idx_s, idx_d, rows0, rows1, rp0, rp1, rq0, rq1,
                 isem, g0, g1, w0, w1):
    wid = lax.axis_index("s") * NC + lax.axis_index("c")
    base = wid * (SLAB // NW)

    pltpu.async_copy(src_hbm.at[wid], idx_s, isem).wait()
    pltpu.async_copy(dst_hbm.at[wid], idx_d, isem).wait()

    def gathers(i, rows, rp, rq, sem):
        c1 = pltpu.async_copy(sender_hbm.at[idx_s.at[i]], rows, sem)
        c2 = pltpu.async_copy(pos_hbm.at[idx_s.at[i]], rp, sem)
        c3 = pltpu.async_copy(ppos_hbm.at[idx_d.at[i]], rq, sem)
        return c1, c2, c3

    def writes(i, rows, rp, rq, sem):
        off = base + i * CHUNK
        pltpu.async_copy(rows, gath_hbm.at[pl.ds(off, CHUNK)], sem)
        pltpu.async_copy(rp, psrc_hbm.at[pl.ds(off, CHUNK)], sem)
        pltpu.async_copy(rq, pdst_hbm.at[pl.ds(off, CHUNK)], sem)

    def drain_writes(rows, rp, rq, sem):
        pltpu.make_async_copy(rows, gath_hbm.at[pl.ds(base, CHUNK)], sem).wait()
        pltpu.make_async_copy(rp, psrc_hbm.at[pl.ds(base, CHUNK)], sem).wait()
        pltpu.make_async_copy(rq, pdst_hbm.at[pl.ds(base, CHUNK)], sem).wait()

    def drain(cs):
        for c in cs:
            c.wait()

    def step(j, _):
        a = 2 * j
        b = a + 1

        @pl.when(j > 0)
        def _():
            drain_writes(rows0, rp0, rq0, w0)
            drain_writes(rows1, rp1, rq1, w1)

        ca = gathers(a, rows0, rp0, rq0, g0)
        cb = gathers(b, rows1, rp1, rq1, g1)
        drain(ca)
        writes(a, rows0, rp0, rq0, w0)
        drain(cb)
        writes(b, rows1, rp1, rq1, w1)
        return 0

    lax.fori_loop(0, GCH // 2, step, 0)
    drain_writes(rows0, rp0, rq0, w0)
    drain_writes(rows1, rp1, rq1, w1)


def _gather(src, dst, sender, pospad, ppospad):
    f = pl.kernel(
        _gather_body,
        out_type=(
            jax.ShapeDtypeStruct((SLAB, D), jnp.float32),
            jax.ShapeDtypeStruct((SLAB, 16), jnp.float32),
            jax.ShapeDtypeStruct((SLAB, 16), jnp.float32),
        ),
        mesh=_sc_mesh(),
        scratch_types=[
            pltpu.VMEM((GCH, CHUNK), jnp.int32),
            pltpu.VMEM((GCH, CHUNK), jnp.int32),
            pltpu.VMEM((CHUNK, D), jnp.float32),
            pltpu.VMEM((CHUNK, D), jnp.float32),
            pltpu.VMEM((CHUNK, 16), jnp.float32),
            pltpu.VMEM((CHUNK, 16), jnp.float32),
            pltpu.VMEM((CHUNK, 16), jnp.float32),
            pltpu.VMEM((CHUNK, 16), jnp.float32),
            pltpu.SemaphoreType.DMA,
            pltpu.SemaphoreType.DMA,
            pltpu.SemaphoreType.DMA,
            pltpu.SemaphoreType.DMA,
            pltpu.SemaphoreType.DMA,
        ],
        compiler_params=pltpu.CompilerParams(use_tc_tiling_on_sc=False),
    )
    return f(src, dst, sender, pospad, ppospad)


def _edge_body(gath_ref, psrc_ref, pdst_ref, ped_ref, c_ref, bc_ref,
               w1_ref, b1_ref, w2s_ref, b2s_ref, w2v_ref, b2v_ref,
               ms_ref, mv_ref):
    g = gath_ref[...]
    disp = jnp.dot(ped_ref[...], c_ref[...], preferred_element_type=jnp.float32)
    vec = pdst_ref[...] - psrc_ref[...] - disp
    l2 = jnp.sum(vec * vec, axis=1, keepdims=True) + 1e-12
    length = jnp.sqrt(l2)
    unit = vec / length

    values = bc_ref[0:1, :]
    inv_std = bc_ref[1:2, :]
    mean = bc_ref[2:3, :]
    diff = (length - values) * INV_STEP
    basis = jnp.exp(-diff * diff) * (1.0 / 1.12)
    bn = (basis - mean) * inv_std

    h = jnp.dot(bn, w1_ref[...], preferred_element_type=jnp.float32) + b1_ref[...]
    h = h * (1.0 / (1.0 + jnp.exp(-h)))
    ws = jnp.dot(h, w2s_ref[...], preferred_element_type=jnp.float32) + b2s_ref[...]
    wv = jnp.dot(h, w2v_ref[...], preferred_element_type=jnp.float32) + b2v_ref[...]

    ms_ref[...] = ws * g
    m = wv * g[:, :DV]
    ux = unit[:, 0:1]
    uy = unit[:, 1:2]
    uz = unit[:, 2:3]
    mv_ref[...] = jnp.concatenate(
        [m * ux, m * uy, m * uz, jnp.zeros_like(m)], axis=1)


def _edge_stage(gath, psrc, pdst, ped16, c16, bconst, w1, b1, w2s, b2s, w2v, b2v):
    nb = SLAB // BE
    blk = lambda r, c: pl.BlockSpec((r, c), lambda i: (i, 0))
    full = lambda r, c: pl.BlockSpec((r, c), lambda i: (0, 0))
    return pl.pallas_call(
        _edge_body,
        grid=(nb,),
        in_specs=[
            blk(BE, D), blk(BE, 16), blk(BE, 16), blk(BE, 16),
            full(16, 16), full(8, 16),
            full(16, HID), full(1, HID),
            full(HID, D), full(1, D),
            full(HID, DV), full(1, DV),
        ],
        out_specs=[blk(BE, D), blk(BE, D)],
        out_shape=[
            jax.ShapeDtypeStruct((SLAB, D), jnp.float32),
            jax.ShapeDtypeStruct((SLAB, D), jnp.float32),
        ],
    )(gath, psrc, pdst, ped16, c16, bconst, w1, b1, w2s, b2s, w2v, b2v)


def _scatter_body(*refs):
    dsts_list = refs[0:NSLAB]
    ms_list = refs[NSLAB:2 * NSLAB]
    mv_list = refs[2 * NSLAB:3 * NSLAB]
    zero_hbm = refs[3 * NSLAB]
    outs_hbm = refs[3 * NSLAB + 1]
    outv_hbm = refs[3 * NSLAB + 2]
    idx, rows0, rows1, acc, isem, l0, l1, a0, a1 = refs[3 * NSLAB + 3:]

    c = lax.axis_index("c")
    s = lax.axis_index("s")

    @pl.when(s == 0)
    def _():
        pltpu.sync_copy(zero_hbm, acc)

    plsc.subcore_barrier()

    base = s * (SLAB // NS)

    for k in range(NSLAB):
        ms_hbm = ms_list[k]
        mv_hbm = mv_list[k]

        pltpu.async_copy(dsts_list[k].at[s], idx, isem).wait()

        def load(i, rows, sem):
            off = base + i * CHUNK

            @pl.when(c == 0)
            def _():
                pltpu.async_copy(ms_hbm.at[pl.ds(off, CHUNK)], rows, sem)

            @pl.when(c == 1)
            def _():
                pltpu.async_copy(mv_hbm.at[pl.ds(off, CHUNK)], rows, sem)

        def wait_load(rows, sem):
            pltpu.make_async_copy(ms_hbm.at[pl.ds(base, CHUNK)], rows, sem).wait()

        load(0, rows0, l0)
        load(1, rows1, l1)

        def step(j, _):
            a = 2 * j
            b = a + 1
            wait_load(rows0, l0)
            ca = pltpu.async_copy(rows0, acc.at[idx.at[a]], a0, add=True)
            wait_load(rows1, l1)
            cb = pltpu.async_copy(rows1, acc.at[idx.at[b]], a1, add=True)
            ca.wait()

            @pl.when(a + 2 < SCH)
            def _():
                load(a + 2, rows0, l0)

            cb.wait()

            @pl.when(b + 2 < SCH)
            def _():
                load(b + 2, rows1, l1)

            return 0

        lax.fori_loop(0, SCH // 2, step, 0)

    plsc.subcore_barrier()

    @pl.when((s == 0) & (c == 0))
    def _():
        pltpu.sync_copy(acc.at[pl.ds(0, P)], outs_hbm)

    @pl.when((s == 0) & (c == 1))
    def _():
        pltpu.sync_copy(acc.at[pl.ds(0, P)], outv_hbm)


def _scatter(dsts_list, ms_list, mv_list, zero):
    f = pl.kernel(
        _scatter_body,
        out_type=(
            jax.ShapeDtypeStruct((P, D), jnp.float32),
            jax.ShapeDtypeStruct((P, D), jnp.float32),
        ),
        mesh=_sc_mesh(),
        scratch_types=[
            pltpu.VMEM((SCH, CHUNK), jnp.int32),
            pltpu.VMEM((CHUNK, D), jnp.float32),
            pltpu.VMEM((CHUNK, D), jnp.float32),
            pltpu.VMEM_SHARED((P_ACC, D), jnp.float32),
            pltpu.SemaphoreType.DMA,
            pltpu.SemaphoreType.DMA,
            pltpu.SemaphoreType.DMA,
            pltpu.SemaphoreType.DMA,
            pltpu.SemaphoreType.DMA,
        ],
    )
    return f(*dsts_list, *ms_list, *mv_list, zero)


def _readout_body(s_ref, v_ref, wos_ref, wov_ref, o_ref):
    ps = s_ref[...] * INV_SQRT_NN
    vx = v_ref[:, 0:DV] * INV_SQRT_NN
    vy = v_ref[:, DV:2 * DV] * INV_SQRT_NN
    vz = v_ref[:, 2 * DV:3 * DV] * INV_SQRT_NN
    vnorm = jnp.sqrt(vx * vx + vy * vy + vz * vz + 1e-12)
    o_ref[...] = (jnp.dot(ps, wos_ref[...], preferred_element_type=jnp.float32)
                  + jnp.dot(vnorm, wov_ref[...], preferred_element_type=jnp.float32))


def _readout(accs, accv, wos, wov):
    return pl.pallas_call(
        _readout_body,
        grid=(P // BP,),
        in_specs=[
            pl.BlockSpec((BP, D), lambda i: (i, 0)),
            pl.BlockSpec((BP, D), lambda i: (i, 0)),
            pl.BlockSpec((D, D), lambda i: (0, 0)),
            pl.BlockSpec((DV, D), lambda i: (0, 0)),
        ],
        out_specs=pl.BlockSpec((BP, D), lambda i: (i, 0)),
        out_shape=jax.ShapeDtypeStruct((P, D), jnp.float32),
    )(accs, accv, wos, wov)


def kernel(atom_representation, positions, positions_probe, cells, probe_edges,
           probe_edges_displacement, splits, W_lin1, W_fc1, b_fc1, W_fc2, b_fc2,
           W_out_s, W_out_v):
    pad = E_PAD - E
    src = jnp.pad(probe_edges[:, 0].astype(jnp.int32), (0, pad))
    dst = probe_edges[:, 1].astype(jnp.int32)
    dst_g = jnp.pad(dst, (0, pad))
    dst_s = jnp.pad(dst, (0, pad), constant_values=DUMMY)

    src_sl = src.reshape(NSLAB, NW, GCH, CHUNK)
    dstg_sl = dst_g.reshape(NSLAB, NW, GCH, CHUNK)
    dsts_sl = dst_s.reshape(NSLAB, NS, SCH, CHUNK)

    pospad = jnp.pad(positions, ((0, 0), (0, 13)))
    ppospad = jnp.pad(positions_probe, ((0, 0), (0, 13)))
    ped16 = jnp.pad(probe_edges_displacement, ((0, pad), (0, 13)))
    c16 = jnp.pad(cells[0], ((0, 13), (0, 13)))
    bconst = jnp.asarray(_BCONST)

    w1 = jnp.pad(W_fc1, ((0, 6), (0, 0)))
    b1 = b_fc1[None, :]
    w2s = W_fc2[:, :D]
    b2s = b_fc2[None, :D]
    w2v = W_fc2[:, D:]
    b2v = b_fc2[None, D:]
    zero = jnp.zeros((P_ACC, D), jnp.float32)

    sender = _lin1(atom_representation, W_lin1)

    ms_list, mv_list, dsts_list = [], [], []
    for k in range(NSLAB):
        gath, psrc, pdst = _gather(src_sl[k], dstg_sl[k], sender, pospad, ppospad)
        ms, mv = _edge_stage(gath, psrc, pdst,
                             ped16[k * SLAB:(k + 1) * SLAB], c16, bconst,
                             w1, b1, w2s, b2s, w2v, b2v)
        ms_list.append(ms)
        mv_list.append(mv)
        dsts_list.append(dsts_sl[k])

    accs, accv = _scatter(dsts_list, ms_list, mv_list, zero)
    return _readout(accs, accv, W_out_s, W_out_v)
